# baseline (speedup 1.0000x reference)
import functools

import jax
import jax.numpy as jnp
from jax import lax
from jax.experimental import pallas as pl
from jax.experimental.pallas import tpu as pltpu
from jax.experimental.pallas import tpu_sc as plsc

MIN_NORM = 1e-15
EPS = 4e-3
MAX_NORM = 1e6

N_NODES = 10000
N_EDGES = 320000
D_IN = 128
D_HID = 256
D_OUT = 40

ROW_BLK = 1000


def _sinhc(t, t2):
    e = jnp.exp(t)
    return jnp.where(t < 1e-4, 1.0 + t2 / 6.0, (e - 1.0 / e) / (2.0 * t))


def _cosh(t):
    e = jnp.exp(t)
    return 0.5 * (e + 1.0 / e)


def _acosh(t):
    return jnp.log(t + jnp.sqrt((t - 1.0) * (t + 1.0)))


def _encoder_body(x_ref, w_ref, b_ref, lo_ref, hi_ref):
    x = x_ref[...]
    q = jnp.sum(x * x, axis=-1, keepdims=True)
    xn = jnp.sqrt(jnp.clip(q, MIN_NORM, None))
    s = _sinhc(xn, xn * xn)
    t1 = jnp.sqrt(jnp.clip(1.0 + s * s * q, EPS, None))
    yn = jnp.sqrt(jnp.clip(s * s * q, MIN_NORM, None))
    theta = jnp.clip(t1, 1.0 + EPS, None)
    g = _acosh(theta) * s / yn
    u = g * x
    M = jnp.dot(u, w_ref[...], preferred_element_type=jnp.float32)

    col = lax.broadcasted_iota(jnp.int32, (1, D_HID), 1)
    is0 = col == 0

    Mv = jnp.where(is0, 0.0, M)
    q2 = jnp.sum(Mv * Mv, axis=-1, keepdims=True)
    xn2 = jnp.sqrt(jnp.clip(q2, MIN_NORM, None))
    s2 = _sinhc(xn2, xn2 * xn2)
    rv = s2 * Mv
    r0 = jnp.sqrt(jnp.clip(1.0 + s2 * s2 * q2, EPS, None))

    b = b_ref[...]
    bv = jnp.where(is0, 0.0, b)
    qb = jnp.sum(bv * bv, axis=-1, keepdims=True)
    bn = jnp.sqrt(jnp.clip(qb, MIN_NORM, None))
    sb = _sinhc(bn, bn * bn)
    ybv = sb * bv
    tb = jnp.sqrt(jnp.clip(1.0 + sb * sb * qb, EPS, None))
    ybn = jnp.sqrt(jnp.clip(sb * sb * qb, MIN_NORM, None))
    thb = jnp.clip(tb, 1.0 + EPS, None)
    ubv = _acosh(thb) * ybv / ybn

    yn_r = jnp.sqrt(jnp.clip(jnp.sum(rv * rv, axis=-1, keepdims=True),
                             MIN_NORM, None))
    yhat = rv / yn_r
    alpha = jnp.sum(yhat * ubv, axis=-1, keepdims=True)
    w0 = alpha * yn_r
    wv = ubv - alpha * (1.0 - r0) * yhat
    ux = jnp.sum(rv * wv, axis=-1, keepdims=True)
    v0 = ux / jnp.clip(r0, EPS, None)
    del w0
    dot = jnp.sum(wv * wv, axis=-1, keepdims=True) + v0 * v0 - 2.0 * v0 * v0
    normu = jnp.sqrt(jnp.clip(dot, EPS, None))
    normu = jnp.clip(normu, None, MAX_NORM)
    th = jnp.clip(normu, MIN_NORM, None)
    ch, shth = _cosh(th), _sinhc(th, th * th)
    z0 = ch * r0 + shth * v0
    zv = ch * rv + shth * wv
    del z0
    qz = jnp.sum(zv * zv, axis=-1, keepdims=True)
    tz = jnp.sqrt(jnp.clip(1.0 + qz, EPS, None))
    zn = jnp.sqrt(jnp.clip(qz, MIN_NORM, None))
    thz = jnp.clip(tz, 1.0 + EPS, None)
    xt = _acosh(thz) * zv / zn
    lo_ref[...] = xt[:, :D_IN]
    hi_ref[...] = xt[:, D_IN:]


def _encoder(x, W1s, b1row):
    return pl.pallas_call(
        _encoder_body,
        grid=(N_NODES // ROW_BLK,),
        in_specs=[
            pl.BlockSpec((ROW_BLK, D_IN), lambda i: (i, 0)),
            pl.BlockSpec((D_IN, D_HID), lambda i: (0, 0)),
            pl.BlockSpec((1, D_HID), lambda i: (0, 0)),
        ],
        out_specs=[pl.BlockSpec((ROW_BLK, D_IN), lambda i: (i, 0)),
                   pl.BlockSpec((ROW_BLK, D_IN), lambda i: (i, 0))],
        out_shape=[jax.ShapeDtypeStruct((N_NODES, D_IN), jnp.float32),
                   jax.ShapeDtypeStruct((N_NODES, D_IN), jnp.float32)],
    )(x, W1s, b1row)


NUM_CORES = 2
NUM_SUBCORES = 16
CHUNK = 128
CPT = 160
E_PAD = CPT * CHUNK * NUM_SUBCORES
ACC_ROWS = 10112
ZERO_PER_TILE = ACC_ROWS // NUM_SUBCORES
OUT_PER_TILE = ACC_ROWS // NUM_SUBCORES


def _segsum_sc(xt_lo, xt_hi, dst2d, src2d, zeros_hbm):
    mesh = plsc.VectorSubcoreMesh(core_axis_name="c", subcore_axis_name="s")

    @functools.partial(
        pl.kernel,
        mesh=mesh,
        out_type=jax.ShapeDtypeStruct((2, ACC_ROWS, D_IN), jnp.float32),
        scratch_types=[
            pltpu.VMEM((CHUNK,), jnp.int32),
            pltpu.VMEM((CHUNK,), jnp.int32),
            pltpu.VMEM((CHUNK, D_IN), jnp.float32),
            pltpu.VMEM_SHARED((ACC_ROWS, D_IN), jnp.float32),
            pltpu.SemaphoreType.DMA,
        ],
    )
    def k(lo_hbm, hi_hbm, dst_hbm, src_hbm, z_hbm, out_hbm, src_v, dst_v,
          rows, acc, gsem):
        c = lax.axis_index("c")
        s = lax.axis_index("s")
        base = s * CPT * CHUNK

        pltpu.sync_copy(z_hbm.at[pl.ds(0, ZERO_PER_TILE)],
                        acc.at[pl.ds(s * ZERO_PER_TILE, ZERO_PER_TILE)])
        plsc.subcore_barrier()

        def run(table_hbm):
            def body(g):
                off = base + g * CHUNK
                pltpu.sync_copy(src_hbm.at[pl.ds(off, CHUNK)], src_v)
                pltpu.sync_copy(dst_hbm.at[pl.ds(off, CHUNK)], dst_v)
                pltpu.async_copy(table_hbm.at[src_v], rows, gsem).wait()
                pltpu.sync_copy(rows, acc.at[dst_v], add=True)

            pl.loop(0, CPT)(body)

        @pl.when(c == 0)
        def _():
            run(lo_hbm)

        @pl.when(c == 1)
        def _():
            run(hi_hbm)

        plsc.subcore_barrier()
        pltpu.sync_copy(acc.at[pl.ds(s * OUT_PER_TILE, OUT_PER_TILE)],
                        out_hbm.at[c, pl.ds(s * OUT_PER_TILE, OUT_PER_TILE)])

    return k(xt_lo, xt_hi, dst2d, src2d, zeros_hbm)


def _decoder_body(lo_ref, hi_ref, wd_ref, bd_ref, out_ref):
    sv = jnp.concatenate([lo_ref[0], hi_ref[0]], axis=-1)
    q3 = jnp.sum(sv * sv, axis=-1, keepdims=True)
    sn = jnp.sqrt(jnp.clip(q3, MIN_NORM, None))
    s3 = _sinhc(sn, sn * sn)
    y = s3 * sv
    t3 = jnp.sqrt(jnp.clip(1.0 + s3 * s3 * q3, EPS, None))
    yn = jnp.sqrt(jnp.clip(s3 * s3 * q3, MIN_NORM, None))
    th3 = jnp.clip(t3, 1.0 + EPS, None)
    l = _acosh(th3) * y / yn
    lp = jnp.maximum(l, 0.0)
    q4 = jnp.sum(lp * lp, axis=-1, keepdims=True)
    ln = jnp.sqrt(jnp.clip(q4, MIN_NORM, None))
    s4 = _sinhc(ln, ln * ln)
    y4 = s4 * lp
    t4 = jnp.sqrt(jnp.clip(1.0 + s4 * s4 * q4, EPS, None))
    y4n = jnp.sqrt(jnp.clip(s4 * s4 * q4, MIN_NORM, None))
    th4 = jnp.clip(t4, 1.0 + EPS, None)
    hd = _acosh(th4) * y4 / y4n
    out = jnp.dot(hd, wd_ref[...], preferred_element_type=jnp.float32)
    out = out + bd_ref[...]
    col = lax.broadcasted_iota(jnp.int32, (1, 128), 1)
    valid = col < D_OUT
    neg = jnp.float32(-1e30)
    m = jnp.max(jnp.where(valid, out, neg), axis=-1, keepdims=True)
    e = jnp.where(valid, jnp.exp(out - m), 0.0)
    lse = jnp.log(jnp.sum(e, axis=-1, keepdims=True))
    out_ref[...] = out - m - lse


def _decoder(sup2, WdT_pad, bd_pad):
    return pl.pallas_call(
        _decoder_body,
        grid=(N_NODES // ROW_BLK,),
        in_specs=[
            pl.BlockSpec((1, ROW_BLK, D_IN), lambda i: (0, i, 0)),
            pl.BlockSpec((1, ROW_BLK, D_IN), lambda i: (1, i, 0)),
            pl.BlockSpec((D_HID, 128), lambda i: (0, 0)),
            pl.BlockSpec((1, 128), lambda i: (0, 0)),
        ],
        out_specs=pl.BlockSpec((ROW_BLK, 128), lambda i: (i, 0)),
        out_shape=jax.ShapeDtypeStruct((N_NODES, 128), jnp.float32),
    )(sup2, sup2, WdT_pad, bd_pad)


def kernel(x, edge_index, W1, b1, Wd, bd):
    W1s = W1[:, 1:].T
    b1row = b1.reshape(1, D_HID)
    WdT_pad = jnp.zeros((D_HID, 128), jnp.float32).at[:, :D_OUT].set(Wd.T)
    bd_pad = jnp.zeros((1, 128), jnp.float32).at[0, :D_OUT].set(bd)
    zeros_hbm = jnp.zeros((ZERO_PER_TILE, D_IN), jnp.float32)

    npad = E_PAD - N_EDGES
    dstp = jnp.concatenate(
        [edge_index[0], N_NODES + (jnp.arange(npad, dtype=jnp.int32)
                                   % (ACC_ROWS - N_NODES))])
    srcp = jnp.concatenate([edge_index[1], jnp.zeros((npad,), jnp.int32)])

    xt_lo, xt_hi = _encoder(x, W1s, b1row)
    sup2 = _segsum_sc(xt_lo, xt_hi, dstp, srcp, zeros_hbm)
    out = _decoder(sup2, WdT_pad, bd_pad)
    return out[:, :D_OUT]


# speedup vs baseline: 1.5848x; 1.5848x over previous
import functools

import jax
import jax.numpy as jnp
from jax import lax
from jax.experimental import pallas as pl
from jax.experimental.pallas import tpu as pltpu
from jax.experimental.pallas import tpu_sc as plsc

MIN_NORM = 1e-15
EPS = 4e-3
MAX_NORM = 1e6

N_NODES = 10000
N_EDGES = 320000
D_IN = 128
D_HID = 256
D_OUT = 40

ROW_BLK = 1000


def _sinhc(t, t2):
    e = jnp.exp(t)
    return jnp.where(t < 1e-4, 1.0 + t2 / 6.0, (e - 1.0 / e) / (2.0 * t))


def _cosh(t):
    e = jnp.exp(t)
    return 0.5 * (e + 1.0 / e)


def _acosh(t):
    return jnp.log(t + jnp.sqrt((t - 1.0) * (t + 1.0)))


def _encoder_body(x_ref, w_ref, b_ref, out_ref):
    x = x_ref[...]
    q = jnp.sum(x * x, axis=-1, keepdims=True)
    xn = jnp.sqrt(jnp.clip(q, MIN_NORM, None))
    s = _sinhc(xn, xn * xn)
    t1 = jnp.sqrt(jnp.clip(1.0 + s * s * q, EPS, None))
    yn = jnp.sqrt(jnp.clip(s * s * q, MIN_NORM, None))
    theta = jnp.clip(t1, 1.0 + EPS, None)
    g = _acosh(theta) * s / yn
    u = g * x
    M = jnp.dot(u, w_ref[...], preferred_element_type=jnp.float32)

    col = lax.broadcasted_iota(jnp.int32, (1, D_HID), 1)
    is0 = col == 0

    Mv = jnp.where(is0, 0.0, M)
    q2 = jnp.sum(Mv * Mv, axis=-1, keepdims=True)
    xn2 = jnp.sqrt(jnp.clip(q2, MIN_NORM, None))
    s2 = _sinhc(xn2, xn2 * xn2)
    rv = s2 * Mv
    r0 = jnp.sqrt(jnp.clip(1.0 + s2 * s2 * q2, EPS, None))

    b = b_ref[...]
    bv = jnp.where(is0, 0.0, b)
    qb = jnp.sum(bv * bv, axis=-1, keepdims=True)
    bn = jnp.sqrt(jnp.clip(qb, MIN_NORM, None))
    sb = _sinhc(bn, bn * bn)
    ybv = sb * bv
    tb = jnp.sqrt(jnp.clip(1.0 + sb * sb * qb, EPS, None))
    ybn = jnp.sqrt(jnp.clip(sb * sb * qb, MIN_NORM, None))
    thb = jnp.clip(tb, 1.0 + EPS, None)
    ubv = _acosh(thb) * ybv / ybn

    yn_r = jnp.sqrt(jnp.clip(jnp.sum(rv * rv, axis=-1, keepdims=True),
                             MIN_NORM, None))
    yhat = rv / yn_r
    alpha = jnp.sum(yhat * ubv, axis=-1, keepdims=True)
    w0 = alpha * yn_r
    wv = ubv - alpha * (1.0 - r0) * yhat
    ux = jnp.sum(rv * wv, axis=-1, keepdims=True)
    v0 = ux / jnp.clip(r0, EPS, None)
    del w0
    dot = jnp.sum(wv * wv, axis=-1, keepdims=True) + v0 * v0 - 2.0 * v0 * v0
    normu = jnp.sqrt(jnp.clip(dot, EPS, None))
    normu = jnp.clip(normu, None, MAX_NORM)
    th = jnp.clip(normu, MIN_NORM, None)
    ch, shth = _cosh(th), _sinhc(th, th * th)
    z0 = ch * r0 + shth * v0
    zv = ch * rv + shth * wv
    del z0
    qz = jnp.sum(zv * zv, axis=-1, keepdims=True)
    tz = jnp.sqrt(jnp.clip(1.0 + qz, EPS, None))
    zn = jnp.sqrt(jnp.clip(qz, MIN_NORM, None))
    thz = jnp.clip(tz, 1.0 + EPS, None)
    xt = _acosh(thz) * zv / zn
    out_ref[0, :, :] = xt[:, :D_IN]
    out_ref[1, :, :] = xt[:, D_IN:]


def _encoder(x, W1s, b1row):
    return pl.pallas_call(
        _encoder_body,
        grid=(N_NODES // ROW_BLK,),
        in_specs=[
            pl.BlockSpec((ROW_BLK, D_IN), lambda i: (i, 0)),
            pl.BlockSpec((D_IN, D_HID), lambda i: (0, 0)),
            pl.BlockSpec((1, D_HID), lambda i: (0, 0)),
        ],
        out_specs=pl.BlockSpec((2, ROW_BLK, D_IN), lambda i: (0, i, 0)),
        out_shape=jax.ShapeDtypeStruct((2, N_NODES, D_IN), jnp.float32),
    )(x, W1s, b1row)


NUM_CORES = 2
NUM_SUBCORES = 16
CHUNK = 128
EDGES_PER_TILE = N_EDGES // NUM_SUBCORES
N_CHUNKS = EDGES_PER_TILE // CHUNK
TAIL = EDGES_PER_TILE - N_CHUNKS * CHUNK
ACC_ROWS = 10112
ZERO_PER_TILE = ACC_ROWS // NUM_SUBCORES
OUT_PER_TILE = ACC_ROWS // NUM_SUBCORES


def _segsum_sc(xt_flat, dst_ids, src_ids, zeros_hbm):
    mesh = plsc.VectorSubcoreMesh(core_axis_name="c", subcore_axis_name="s")

    @functools.partial(
        pl.kernel,
        mesh=mesh,
        out_type=jax.ShapeDtypeStruct((2, ACC_ROWS, D_IN), jnp.float32),
        scratch_types=[
            pltpu.VMEM((CHUNK,), jnp.int32),
            pltpu.VMEM((CHUNK,), jnp.int32),
            pltpu.VMEM((CHUNK, D_IN), jnp.float32),
            pltpu.VMEM_SHARED((ACC_ROWS, D_IN), jnp.float32),
            pltpu.SemaphoreType.DMA,
        ],
    )
    def k(xt_hbm, dst_hbm, src_hbm, z_hbm, out_hbm, src_v, dst_v, rows, acc,
          gsem):
        c = lax.axis_index("c")
        s = lax.axis_index("s")
        row_off = c * N_NODES
        offv = jnp.full((16,), 1, jnp.int32) * row_off

        pltpu.sync_copy(z_hbm.at[pl.ds(0, ZERO_PER_TILE)],
                        acc.at[pl.ds(s * ZERO_PER_TILE, ZERO_PER_TILE)])
        plsc.subcore_barrier()

        base = s * EDGES_PER_TILE

        def do_chunk(off, count):
            pltpu.sync_copy(src_hbm.at[pl.ds(off, count)],
                            src_v.at[pl.ds(0, count)])
            pltpu.sync_copy(dst_hbm.at[pl.ds(off, count)],
                            dst_v.at[pl.ds(0, count)])
            for j in range(count // 16):
                sl = pl.ds(j * 16, 16)
                src_v[sl] = src_v[sl] + offv
            pltpu.async_copy(xt_hbm.at[src_v.at[pl.ds(0, count)]],
                             rows.at[pl.ds(0, count)], gsem).wait()
            pltpu.sync_copy(rows.at[pl.ds(0, count)],
                            acc.at[dst_v.at[pl.ds(0, count)]], add=True)

        def body(g):
            do_chunk(base + g * CHUNK, CHUNK)

        pl.loop(0, N_CHUNKS)(body)
        if TAIL:
            do_chunk(base + N_CHUNKS * CHUNK, TAIL)

        plsc.subcore_barrier()
        pltpu.sync_copy(acc.at[pl.ds(s * OUT_PER_TILE, OUT_PER_TILE)],
                        out_hbm.at[c, pl.ds(s * OUT_PER_TILE, OUT_PER_TILE)])

    return k(xt_flat, dst_ids, src_ids, zeros_hbm)


def _decoder_body(lo_ref, hi_ref, wd_ref, bd_ref, out_ref):
    sv = jnp.concatenate([lo_ref[0], hi_ref[0]], axis=-1)
    q3 = jnp.sum(sv * sv, axis=-1, keepdims=True)
    sn = jnp.sqrt(jnp.clip(q3, MIN_NORM, None))
    s3 = _sinhc(sn, sn * sn)
    y = s3 * sv
    t3 = jnp.sqrt(jnp.clip(1.0 + s3 * s3 * q3, EPS, None))
    yn = jnp.sqrt(jnp.clip(s3 * s3 * q3, MIN_NORM, None))
    th3 = jnp.clip(t3, 1.0 + EPS, None)
    l = _acosh(th3) * y / yn
    lp = jnp.maximum(l, 0.0)
    q4 = jnp.sum(lp * lp, axis=-1, keepdims=True)
    ln = jnp.sqrt(jnp.clip(q4, MIN_NORM, None))
    s4 = _sinhc(ln, ln * ln)
    y4 = s4 * lp
    t4 = jnp.sqrt(jnp.clip(1.0 + s4 * s4 * q4, EPS, None))
    y4n = jnp.sqrt(jnp.clip(s4 * s4 * q4, MIN_NORM, None))
    th4 = jnp.clip(t4, 1.0 + EPS, None)
    hd = _acosh(th4) * y4 / y4n
    out = jnp.dot(hd, wd_ref[...], preferred_element_type=jnp.float32)
    out = out + bd_ref[...]
    col = lax.broadcasted_iota(jnp.int32, (1, 128), 1)
    valid = col < D_OUT
    neg = jnp.float32(-1e30)
    m = jnp.max(jnp.where(valid, out, neg), axis=-1, keepdims=True)
    e = jnp.where(valid, jnp.exp(out - m), 0.0)
    lse = jnp.log(jnp.sum(e, axis=-1, keepdims=True))
    out_ref[...] = out - m - lse


def _decoder(sup2, WdT_pad, bd_pad):
    return pl.pallas_call(
        _decoder_body,
        grid=(N_NODES // ROW_BLK,),
        in_specs=[
            pl.BlockSpec((1, ROW_BLK, D_IN), lambda i: (0, i, 0)),
            pl.BlockSpec((1, ROW_BLK, D_IN), lambda i: (1, i, 0)),
            pl.BlockSpec((D_HID, 128), lambda i: (0, 0)),
            pl.BlockSpec((1, 128), lambda i: (0, 0)),
        ],
        out_specs=pl.BlockSpec((ROW_BLK, 128), lambda i: (i, 0)),
        out_shape=jax.ShapeDtypeStruct((N_NODES, 128), jnp.float32),
    )(sup2, sup2, WdT_pad, bd_pad)


def kernel(x, edge_index, W1, b1, Wd, bd):
    W1s = W1[:, 1:].T
    b1row = b1.reshape(1, D_HID)
    WdT_pad = jnp.zeros((D_HID, 128), jnp.float32).at[:, :D_OUT].set(Wd.T)
    bd_pad = jnp.zeros((1, 128), jnp.float32).at[0, :D_OUT].set(bd)
    zeros_hbm = jnp.zeros((ZERO_PER_TILE, D_IN), jnp.float32)

    xt2 = _encoder(x, W1s, b1row)
    xt_flat = xt2.reshape(2 * N_NODES, D_IN)
    sup2 = _segsum_sc(xt_flat, edge_index[0], edge_index[1], zeros_hbm)
    out = _decoder(sup2, WdT_pad, bd_pad)
    return out[:, :D_OUT]


# speedup vs baseline: 2.0985x; 1.3242x over previous
import functools

import jax
import jax.numpy as jnp
from jax import lax
from jax.experimental import pallas as pl
from jax.experimental.pallas import tpu as pltpu
from jax.experimental.pallas import tpu_sc as plsc

MIN_NORM = 1e-15
EPS = 4e-3
MAX_NORM = 1e6

N_NODES = 10000
N_EDGES = 320000
D_IN = 128
D_HID = 256
D_OUT = 40

ROW_BLK = 1000


def _sinhc(t, t2):
    e = jnp.exp(t)
    return jnp.where(t < 1e-4, 1.0 + t2 / 6.0, (e - 1.0 / e) / (2.0 * t))


def _cosh(t):
    e = jnp.exp(t)
    return 0.5 * (e + 1.0 / e)


def _acosh(t):
    return jnp.log(t + jnp.sqrt((t - 1.0) * (t + 1.0)))


def _encoder_body(x_ref, w_ref, b_ref, out_ref):
    x = x_ref[...]
    q = jnp.sum(x * x, axis=-1, keepdims=True)
    xn = jnp.sqrt(jnp.clip(q, MIN_NORM, None))
    s = _sinhc(xn, xn * xn)
    t1 = jnp.sqrt(jnp.clip(1.0 + s * s * q, EPS, None))
    yn = jnp.sqrt(jnp.clip(s * s * q, MIN_NORM, None))
    theta = jnp.clip(t1, 1.0 + EPS, None)
    g = _acosh(theta) * s / yn
    u = g * x
    M = jnp.dot(u, w_ref[...], preferred_element_type=jnp.float32)

    col = lax.broadcasted_iota(jnp.int32, (1, D_HID), 1)
    is0 = col == 0

    Mv = jnp.where(is0, 0.0, M)
    q2 = jnp.sum(Mv * Mv, axis=-1, keepdims=True)
    xn2 = jnp.sqrt(jnp.clip(q2, MIN_NORM, None))
    s2 = _sinhc(xn2, xn2 * xn2)
    rv = s2 * Mv
    r0 = jnp.sqrt(jnp.clip(1.0 + s2 * s2 * q2, EPS, None))

    b = b_ref[...]
    bv = jnp.where(is0, 0.0, b)
    qb = jnp.sum(bv * bv, axis=-1, keepdims=True)
    bn = jnp.sqrt(jnp.clip(qb, MIN_NORM, None))
    sb = _sinhc(bn, bn * bn)
    ybv = sb * bv
    tb = jnp.sqrt(jnp.clip(1.0 + sb * sb * qb, EPS, None))
    ybn = jnp.sqrt(jnp.clip(sb * sb * qb, MIN_NORM, None))
    thb = jnp.clip(tb, 1.0 + EPS, None)
    ubv = _acosh(thb) * ybv / ybn

    yn_r = jnp.sqrt(jnp.clip(jnp.sum(rv * rv, axis=-1, keepdims=True),
                             MIN_NORM, None))
    yhat = rv / yn_r
    alpha = jnp.sum(yhat * ubv, axis=-1, keepdims=True)
    w0 = alpha * yn_r
    wv = ubv - alpha * (1.0 - r0) * yhat
    ux = jnp.sum(rv * wv, axis=-1, keepdims=True)
    v0 = ux / jnp.clip(r0, EPS, None)
    del w0
    dot = jnp.sum(wv * wv, axis=-1, keepdims=True) + v0 * v0 - 2.0 * v0 * v0
    normu = jnp.sqrt(jnp.clip(dot, EPS, None))
    normu = jnp.clip(normu, None, MAX_NORM)
    th = jnp.clip(normu, MIN_NORM, None)
    ch, shth = _cosh(th), _sinhc(th, th * th)
    z0 = ch * r0 + shth * v0
    zv = ch * rv + shth * wv
    del z0
    qz = jnp.sum(zv * zv, axis=-1, keepdims=True)
    tz = jnp.sqrt(jnp.clip(1.0 + qz, EPS, None))
    zn = jnp.sqrt(jnp.clip(qz, MIN_NORM, None))
    thz = jnp.clip(tz, 1.0 + EPS, None)
    xt = _acosh(thz) * zv / zn
    out_ref[0, :, :] = xt[:, :D_IN]
    out_ref[1, :, :] = xt[:, D_IN:]


def _encoder(x, W1s, b1row):
    return pl.pallas_call(
        _encoder_body,
        grid=(N_NODES // ROW_BLK,),
        in_specs=[
            pl.BlockSpec((ROW_BLK, D_IN), lambda i: (i, 0)),
            pl.BlockSpec((D_IN, D_HID), lambda i: (0, 0)),
            pl.BlockSpec((1, D_HID), lambda i: (0, 0)),
        ],
        out_specs=pl.BlockSpec((2, ROW_BLK, D_IN), lambda i: (0, i, 0)),
        out_shape=jax.ShapeDtypeStruct((2, N_NODES, D_IN), jnp.float32),
    )(x, W1s, b1row)


NUM_CORES = 2
NUM_SUBCORES = 16
CHUNK = 128
EDGES_PER_TILE = N_EDGES // NUM_SUBCORES
N_CHUNKS = EDGES_PER_TILE // CHUNK
TAIL = EDGES_PER_TILE - N_CHUNKS * CHUNK
ACC_ROWS = 10112
ZERO_PER_TILE = ACC_ROWS // NUM_SUBCORES
OUT_PER_TILE = ACC_ROWS // NUM_SUBCORES


def _segsum_sc(xt_flat, dst_ids, src_ids, zeros_hbm):
    mesh = plsc.VectorSubcoreMesh(core_axis_name="c", subcore_axis_name="s")

    @functools.partial(
        pl.kernel,
        mesh=mesh,
        out_type=jax.ShapeDtypeStruct((2, ACC_ROWS, D_IN), jnp.float32),
        scratch_types=[
            [pltpu.VMEM((CHUNK,), jnp.int32) for _ in range(2)],
            [pltpu.VMEM((CHUNK,), jnp.int32) for _ in range(2)],
            [pltpu.VMEM((CHUNK, D_IN), jnp.float32) for _ in range(2)],
            pltpu.VMEM_SHARED((ACC_ROWS, D_IN), jnp.float32),
            [pltpu.SemaphoreType.DMA for _ in range(2)],
        ],
    )
    def k(xt_hbm, dst_hbm, src_hbm, z_hbm, out_hbm, src_v, dst_v, rows, acc,
          gsem):
        c = lax.axis_index("c")
        s = lax.axis_index("s")
        row_off = c * N_NODES
        offv = jnp.full((16,), 1, jnp.int32) * row_off

        pltpu.sync_copy(z_hbm.at[pl.ds(0, ZERO_PER_TILE)],
                        acc.at[pl.ds(s * ZERO_PER_TILE, ZERO_PER_TILE)])
        plsc.subcore_barrier()

        base = s * EDGES_PER_TILE

        def load_ids(off, count, b):
            pltpu.sync_copy(src_hbm.at[pl.ds(off, count)],
                            src_v[b].at[pl.ds(0, count)])
            pltpu.sync_copy(dst_hbm.at[pl.ds(off, count)],
                            dst_v[b].at[pl.ds(0, count)])
            for j in range(count // 16):
                sl = pl.ds(j * 16, 16)
                src_v[b][sl] = src_v[b][sl] + offv

        def gather(count, b):
            return pltpu.async_copy(xt_hbm.at[src_v[b].at[pl.ds(0, count)]],
                                    rows[b].at[pl.ds(0, count)], gsem[b])

        def scatter(count, b):
            pltpu.sync_copy(rows[b].at[pl.ds(0, count)],
                            acc.at[dst_v[b].at[pl.ds(0, count)]], add=True)

        def body(g):
            off = base + 2 * g * CHUNK
            load_ids(off, CHUNK, 0)
            da = gather(CHUNK, 0)
            load_ids(off + CHUNK, CHUNK, 1)
            db = gather(CHUNK, 1)
            da.wait()
            scatter(CHUNK, 0)
            db.wait()
            scatter(CHUNK, 1)

        pl.loop(0, N_CHUNKS // 2)(body)
        if TAIL:
            load_ids(base + N_CHUNKS * CHUNK, TAIL, 0)
            gather(TAIL, 0).wait()
            scatter(TAIL, 0)

        plsc.subcore_barrier()
        pltpu.sync_copy(acc.at[pl.ds(s * OUT_PER_TILE, OUT_PER_TILE)],
                        out_hbm.at[c, pl.ds(s * OUT_PER_TILE, OUT_PER_TILE)])

    return k(xt_flat, dst_ids, src_ids, zeros_hbm)


def _decoder_body(lo_ref, hi_ref, wd_ref, bd_ref, out_ref):
    sv = jnp.concatenate([lo_ref[0], hi_ref[0]], axis=-1)
    q3 = jnp.sum(sv * sv, axis=-1, keepdims=True)
    sn = jnp.sqrt(jnp.clip(q3, MIN_NORM, None))
    s3 = _sinhc(sn, sn * sn)
    y = s3 * sv
    t3 = jnp.sqrt(jnp.clip(1.0 + s3 * s3 * q3, EPS, None))
    yn = jnp.sqrt(jnp.clip(s3 * s3 * q3, MIN_NORM, None))
    th3 = jnp.clip(t3, 1.0 + EPS, None)
    l = _acosh(th3) * y / yn
    lp = jnp.maximum(l, 0.0)
    q4 = jnp.sum(lp * lp, axis=-1, keepdims=True)
    ln = jnp.sqrt(jnp.clip(q4, MIN_NORM, None))
    s4 = _sinhc(ln, ln * ln)
    y4 = s4 * lp
    t4 = jnp.sqrt(jnp.clip(1.0 + s4 * s4 * q4, EPS, None))
    y4n = jnp.sqrt(jnp.clip(s4 * s4 * q4, MIN_NORM, None))
    th4 = jnp.clip(t4, 1.0 + EPS, None)
    hd = _acosh(th4) * y4 / y4n
    out = jnp.dot(hd, wd_ref[...], preferred_element_type=jnp.float32)
    out = out + bd_ref[...]
    col = lax.broadcasted_iota(jnp.int32, (1, 128), 1)
    valid = col < D_OUT
    neg = jnp.float32(-1e30)
    m = jnp.max(jnp.where(valid, out, neg), axis=-1, keepdims=True)
    e = jnp.where(valid, jnp.exp(out - m), 0.0)
    lse = jnp.log(jnp.sum(e, axis=-1, keepdims=True))
    out_ref[...] = out - m - lse


def _decoder(sup2, WdT_pad, bd_pad):
    return pl.pallas_call(
        _decoder_body,
        grid=(N_NODES // ROW_BLK,),
        in_specs=[
            pl.BlockSpec((1, ROW_BLK, D_IN), lambda i: (0, i, 0)),
            pl.BlockSpec((1, ROW_BLK, D_IN), lambda i: (1, i, 0)),
            pl.BlockSpec((D_HID, 128), lambda i: (0, 0)),
            pl.BlockSpec((1, 128), lambda i: (0, 0)),
        ],
        out_specs=pl.BlockSpec((ROW_BLK, 128), lambda i: (i, 0)),
        out_shape=jax.ShapeDtypeStruct((N_NODES, 128), jnp.float32),
    )(sup2, sup2, WdT_pad, bd_pad)


def kernel(x, edge_index, W1, b1, Wd, bd):
    W1s = W1[:, 1:].T
    b1row = b1.reshape(1, D_HID)
    WdT_pad = jnp.zeros((D_HID, 128), jnp.float32).at[:, :D_OUT].set(Wd.T)
    bd_pad = jnp.zeros((1, 128), jnp.float32).at[0, :D_OUT].set(bd)
    zeros_hbm = jnp.zeros((ZERO_PER_TILE, D_IN), jnp.float32)

    xt2 = _encoder(x, W1s, b1row)
    xt_flat = xt2.reshape(2 * N_NODES, D_IN)
    sup2 = _segsum_sc(xt_flat, edge_index[0], edge_index[1], zeros_hbm)
    out = _decoder(sup2, WdT_pad, bd_pad)
    return out[:, :D_OUT]


# speedup vs baseline: 2.3360x; 1.1132x over previous
import functools

import jax
import jax.numpy as jnp
from jax import lax
from jax.experimental import pallas as pl
from jax.experimental.pallas import tpu as pltpu
from jax.experimental.pallas import tpu_sc as plsc

MIN_NORM = 1e-15
EPS = 4e-3
MAX_NORM = 1e6

N_NODES = 10000
N_EDGES = 320000
D_IN = 128
D_HID = 256
D_OUT = 40

ROW_BLK = 1000


def _sinhc(t, t2):
    e = jnp.exp(t)
    return jnp.where(t < 1e-4, 1.0 + t2 / 6.0, (e - 1.0 / e) / (2.0 * t))


def _cosh(t):
    e = jnp.exp(t)
    return 0.5 * (e + 1.0 / e)


def _acosh(t):
    return jnp.log(t + jnp.sqrt((t - 1.0) * (t + 1.0)))


def _encoder_body(x_ref, w_ref, b_ref, out_ref):
    x = x_ref[...]
    q = jnp.sum(x * x, axis=-1, keepdims=True)
    xn = jnp.sqrt(jnp.clip(q, MIN_NORM, None))
    s = _sinhc(xn, xn * xn)
    t1 = jnp.sqrt(jnp.clip(1.0 + s * s * q, EPS, None))
    yn = jnp.sqrt(jnp.clip(s * s * q, MIN_NORM, None))
    theta = jnp.clip(t1, 1.0 + EPS, None)
    g = _acosh(theta) * s / yn
    u = g * x
    M = jnp.dot(u, w_ref[...], preferred_element_type=jnp.float32)

    col = lax.broadcasted_iota(jnp.int32, (1, D_HID), 1)
    is0 = col == 0

    Mv = jnp.where(is0, 0.0, M)
    q2 = jnp.sum(Mv * Mv, axis=-1, keepdims=True)
    xn2 = jnp.sqrt(jnp.clip(q2, MIN_NORM, None))
    s2 = _sinhc(xn2, xn2 * xn2)
    rv = s2 * Mv
    r0 = jnp.sqrt(jnp.clip(1.0 + s2 * s2 * q2, EPS, None))

    b = b_ref[...]
    bv = jnp.where(is0, 0.0, b)
    qb = jnp.sum(bv * bv, axis=-1, keepdims=True)
    bn = jnp.sqrt(jnp.clip(qb, MIN_NORM, None))
    sb = _sinhc(bn, bn * bn)
    ybv = sb * bv
    tb = jnp.sqrt(jnp.clip(1.0 + sb * sb * qb, EPS, None))
    ybn = jnp.sqrt(jnp.clip(sb * sb * qb, MIN_NORM, None))
    thb = jnp.clip(tb, 1.0 + EPS, None)
    ubv = _acosh(thb) * ybv / ybn

    yn_r = jnp.sqrt(jnp.clip(jnp.sum(rv * rv, axis=-1, keepdims=True),
                             MIN_NORM, None))
    yhat = rv / yn_r
    alpha = jnp.sum(yhat * ubv, axis=-1, keepdims=True)
    w0 = alpha * yn_r
    wv = ubv - alpha * (1.0 - r0) * yhat
    ux = jnp.sum(rv * wv, axis=-1, keepdims=True)
    v0 = ux / jnp.clip(r0, EPS, None)
    del w0
    dot = jnp.sum(wv * wv, axis=-1, keepdims=True) + v0 * v0 - 2.0 * v0 * v0
    normu = jnp.sqrt(jnp.clip(dot, EPS, None))
    normu = jnp.clip(normu, None, MAX_NORM)
    th = jnp.clip(normu, MIN_NORM, None)
    ch, shth = _cosh(th), _sinhc(th, th * th)
    z0 = ch * r0 + shth * v0
    zv = ch * rv + shth * wv
    del z0
    qz = jnp.sum(zv * zv, axis=-1, keepdims=True)
    tz = jnp.sqrt(jnp.clip(1.0 + qz, EPS, None))
    zn = jnp.sqrt(jnp.clip(qz, MIN_NORM, None))
    thz = jnp.clip(tz, 1.0 + EPS, None)
    xt = _acosh(thz) * zv / zn
    out_ref[0, :, :] = xt[:, :D_IN]
    out_ref[1, :, :] = xt[:, D_IN:]


def _encoder(x, W1s, b1row):
    return pl.pallas_call(
        _encoder_body,
        grid=(N_NODES // ROW_BLK,),
        in_specs=[
            pl.BlockSpec((ROW_BLK, D_IN), lambda i: (i, 0)),
            pl.BlockSpec((D_IN, D_HID), lambda i: (0, 0)),
            pl.BlockSpec((1, D_HID), lambda i: (0, 0)),
        ],
        out_specs=pl.BlockSpec((2, ROW_BLK, D_IN), lambda i: (0, i, 0)),
        out_shape=jax.ShapeDtypeStruct((2, N_NODES, D_IN), jnp.float32),
    )(x, W1s, b1row)


NUM_CORES = 2
NUM_SUBCORES = 16
CHUNK = 128
EDGES_PER_TILE = N_EDGES // NUM_SUBCORES
N_CHUNKS = EDGES_PER_TILE // CHUNK
TAIL = EDGES_PER_TILE - N_CHUNKS * CHUNK
ACC_ROWS = 10112
ZERO_PER_TILE = ACC_ROWS // NUM_SUBCORES
OUT_PER_TILE = ACC_ROWS // NUM_SUBCORES


def _segsum_sc(xt_flat, dst_ids, src_ids, zeros_hbm):
    mesh = plsc.VectorSubcoreMesh(core_axis_name="c", subcore_axis_name="s")

    @functools.partial(
        pl.kernel,
        mesh=mesh,
        out_type=jax.ShapeDtypeStruct((2, ACC_ROWS, D_IN), jnp.float32),
        scratch_types=[
            [pltpu.VMEM((CHUNK,), jnp.int32) for _ in range(3)],
            [pltpu.VMEM((CHUNK,), jnp.int32) for _ in range(3)],
            [pltpu.VMEM((CHUNK, D_IN), jnp.float32) for _ in range(3)],
            pltpu.VMEM_SHARED((ACC_ROWS, D_IN), jnp.float32),
            [pltpu.SemaphoreType.DMA for _ in range(3)],
        ],
    )
    def k(xt_hbm, dst_hbm, src_hbm, z_hbm, out_hbm, src_v, dst_v, rows, acc,
          gsem):
        c = lax.axis_index("c")
        s = lax.axis_index("s")
        row_off = c * N_NODES
        offv = jnp.full((16,), 1, jnp.int32) * row_off

        pltpu.sync_copy(z_hbm.at[pl.ds(0, ZERO_PER_TILE)],
                        acc.at[pl.ds(s * ZERO_PER_TILE, ZERO_PER_TILE)])
        plsc.subcore_barrier()

        base = s * EDGES_PER_TILE

        def load_ids(off, count, b):
            pltpu.sync_copy(src_hbm.at[pl.ds(off, count)],
                            src_v[b].at[pl.ds(0, count)])
            pltpu.sync_copy(dst_hbm.at[pl.ds(off, count)],
                            dst_v[b].at[pl.ds(0, count)])
            for j in range(count // 16):
                sl = pl.ds(j * 16, 16)
                src_v[b][sl] = src_v[b][sl] + offv

        def gather(count, b):
            return pltpu.async_copy(xt_hbm.at[src_v[b].at[pl.ds(0, count)]],
                                    rows[b].at[pl.ds(0, count)], gsem[b])

        def scatter(count, b):
            pltpu.sync_copy(rows[b].at[pl.ds(0, count)],
                            acc.at[dst_v[b].at[pl.ds(0, count)]], add=True)

        def body(g):
            off = base + 3 * g * CHUNK
            load_ids(off, CHUNK, 0)
            d0 = gather(CHUNK, 0)
            load_ids(off + CHUNK, CHUNK, 1)
            d1 = gather(CHUNK, 1)
            load_ids(off + 2 * CHUNK, CHUNK, 2)
            d2 = gather(CHUNK, 2)
            d0.wait()
            scatter(CHUNK, 0)
            d1.wait()
            scatter(CHUNK, 1)
            d2.wait()
            scatter(CHUNK, 2)

        pl.loop(0, N_CHUNKS // 3)(body)
        if TAIL:
            load_ids(base + N_CHUNKS * CHUNK, TAIL, 0)
            gather(TAIL, 0).wait()
            scatter(TAIL, 0)

        plsc.subcore_barrier()
        pltpu.sync_copy(acc.at[pl.ds(s * OUT_PER_TILE, OUT_PER_TILE)],
                        out_hbm.at[c, pl.ds(s * OUT_PER_TILE, OUT_PER_TILE)])

    return k(xt_flat, dst_ids, src_ids, zeros_hbm)


def _decoder_body(lo_ref, hi_ref, wd_ref, bd_ref, out_ref):
    sv = jnp.concatenate([lo_ref[0], hi_ref[0]], axis=-1)
    q3 = jnp.sum(sv * sv, axis=-1, keepdims=True)
    sn = jnp.sqrt(jnp.clip(q3, MIN_NORM, None))
    s3 = _sinhc(sn, sn * sn)
    y = s3 * sv
    t3 = jnp.sqrt(jnp.clip(1.0 + s3 * s3 * q3, EPS, None))
    yn = jnp.sqrt(jnp.clip(s3 * s3 * q3, MIN_NORM, None))
    th3 = jnp.clip(t3, 1.0 + EPS, None)
    l = _acosh(th3) * y / yn
    lp = jnp.maximum(l, 0.0)
    q4 = jnp.sum(lp * lp, axis=-1, keepdims=True)
    ln = jnp.sqrt(jnp.clip(q4, MIN_NORM, None))
    s4 = _sinhc(ln, ln * ln)
    y4 = s4 * lp
    t4 = jnp.sqrt(jnp.clip(1.0 + s4 * s4 * q4, EPS, None))
    y4n = jnp.sqrt(jnp.clip(s4 * s4 * q4, MIN_NORM, None))
    th4 = jnp.clip(t4, 1.0 + EPS, None)
    hd = _acosh(th4) * y4 / y4n
    out = jnp.dot(hd, wd_ref[...], preferred_element_type=jnp.float32)
    out = out + bd_ref[...]
    col = lax.broadcasted_iota(jnp.int32, (1, 128), 1)
    valid = col < D_OUT
    neg = jnp.float32(-1e30)
    m = jnp.max(jnp.where(valid, out, neg), axis=-1, keepdims=True)
    e = jnp.where(valid, jnp.exp(out - m), 0.0)
    lse = jnp.log(jnp.sum(e, axis=-1, keepdims=True))
    out_ref[...] = out - m - lse


def _decoder(sup2, WdT_pad, bd_pad):
    return pl.pallas_call(
        _decoder_body,
        grid=(N_NODES // ROW_BLK,),
        in_specs=[
            pl.BlockSpec((1, ROW_BLK, D_IN), lambda i: (0, i, 0)),
            pl.BlockSpec((1, ROW_BLK, D_IN), lambda i: (1, i, 0)),
            pl.BlockSpec((D_HID, 128), lambda i: (0, 0)),
            pl.BlockSpec((1, 128), lambda i: (0, 0)),
        ],
        out_specs=pl.BlockSpec((ROW_BLK, 128), lambda i: (i, 0)),
        out_shape=jax.ShapeDtypeStruct((N_NODES, 128), jnp.float32),
    )(sup2, sup2, WdT_pad, bd_pad)


def kernel(x, edge_index, W1, b1, Wd, bd):
    W1s = W1[:, 1:].T
    b1row = b1.reshape(1, D_HID)
    WdT_pad = jnp.zeros((D_HID, 128), jnp.float32).at[:, :D_OUT].set(Wd.T)
    bd_pad = jnp.zeros((1, 128), jnp.float32).at[0, :D_OUT].set(bd)
    zeros_hbm = jnp.zeros((ZERO_PER_TILE, D_IN), jnp.float32)

    xt2 = _encoder(x, W1s, b1row)
    xt_flat = xt2.reshape(2 * N_NODES, D_IN)
    sup2 = _segsum_sc(xt_flat, edge_index[0], edge_index[1], zeros_hbm)
    out = _decoder(sup2, WdT_pad, bd_pad)
    return out[:, :D_OUT]
